# baseline (speedup 1.0000x reference)
import functools

import jax
import jax.numpy as jnp
from jax import lax
from jax.experimental import pallas as pl
from jax.experimental.pallas import tpu as pltpu
from jax.experimental.pallas import tpu_sc as plsc

N_NODES = 10000
HIDDEN = 256
HALF = 128
VOCAB = 5000
N_TYPES = 4
EDGES_PER_TYPE = 40000
N_TOKENS = 20000
TIME_STEPS = [3, 3]

NC = 2
NS = 16
CHUNK = 64

NODE_P = 10112
ZROWS = NODE_P // NS

E_CHUNKS = 160
TOK_CHUNKS = 32
E_PAD = NS * E_CHUNKS * CHUNK
TOK_PAD = NS * TOK_CHUNKS * CHUNK
NBUF = 4
NIDX = 8

_DUMMY_TGT = N_NODES


@functools.lru_cache(maxsize=None)
def _make_sc_scatter(n_chunks):
  mesh = plsc.VectorSubcoreMesh(core_axis_name="c", subcore_axis_name="s",
                                num_cores=NC, num_subcores=NS)
  per_tile = n_chunks * CHUNK

  @functools.partial(
      pl.kernel,
      out_type=jax.ShapeDtypeStruct((NC, NODE_P, HALF), jnp.float32),
      mesh=mesh,
      scratch_types=[
          pltpu.VMEM((NIDX, CHUNK), jnp.int32),
          pltpu.VMEM((NIDX, CHUNK), jnp.int32),
          pltpu.VMEM((NBUF, CHUNK, HALF), jnp.float32),
          pltpu.VMEM_SHARED((NODE_P, HALF), jnp.float32),
          pltpu.SemaphoreType.DMA((NBUF,)),
          pltpu.SemaphoreType.DMA((NBUF,)),
          pltpu.SemaphoreType.DMA((NIDX,)),
      ],
  )
  def k(table_hbm, src_hbm, tgt_hbm, zero_hbm, out_hbm,
        src_v, tgt_v, rows_v, acc_sh, sem_g, sem_s, sem_i):
    c = lax.axis_index("c")
    s = lax.axis_index("s")
    sbase = (c * NS + s) * per_tile
    tbase = s * per_tile

    def load_idx(j, bi):
      pltpu.async_copy(src_hbm.at[pl.ds(sbase + j * CHUNK, CHUNK)],
                       src_v.at[bi], sem_i.at[bi])
      pltpu.async_copy(tgt_hbm.at[pl.ds(tbase + j * CHUNK, CHUNK)],
                       tgt_v.at[bi], sem_i.at[bi])

    def wait_idx(bi):
      pltpu.make_async_copy(src_hbm.at[pl.ds(sbase, CHUNK)], src_v.at[bi],
                            sem_i.at[bi]).wait()
      pltpu.make_async_copy(tgt_hbm.at[pl.ds(tbase, CHUNK)], tgt_v.at[bi],
                            sem_i.at[bi]).wait()

    def gather(j, b, bi):
      pltpu.async_copy(table_hbm.at[src_v.at[bi]], rows_v.at[b], sem_g.at[b])

    def wait_gather(b):
      pltpu.make_async_copy(table_hbm.at[src_v.at[0]], rows_v.at[b],
                            sem_g.at[b]).wait()

    def scatter(b, bi):
      pltpu.async_copy(rows_v.at[b], acc_sh.at[tgt_v.at[bi]], sem_s.at[b],
                       add=True)

    def wait_scatter(b):
      pltpu.make_async_copy(rows_v.at[b], acc_sh.at[tgt_v.at[0]],
                            sem_s.at[b]).wait()

    for j in range(6):
      load_idx(j, j)
    z0 = s * ZROWS
    pltpu.sync_copy(zero_hbm.at[pl.ds(z0, ZROWS)], acc_sh.at[pl.ds(z0, ZROWS)])
    plsc.subcore_barrier()

    for j in range(2):
      wait_idx(j)
      gather(j, j, j)

    def oct_body(it, carry):
      kk = it * NIDX
      for i in range(NIDX):
        kc = kk + i
        b = i % NBUF
        bi = i
        wait_gather(b)
        scatter(b, bi)
        j = kc + 2
        bj = (i + 2) % NBUF
        bij = (i + 2) % NIDX

        @pl.when(j < n_chunks)
        def _():
          @pl.when(j >= NBUF)
          def _():
            wait_scatter(bj)
          @pl.when(j + 4 < n_chunks)
          def _():
            load_idx(j + 4, (i + 6) % NIDX)
          wait_idx(bij)
          gather(j, bj, bij)
      return carry

    lax.fori_loop(0, n_chunks // NIDX, oct_body, 0)
    for b in range(NBUF):
      wait_scatter(b)
    plsc.subcore_barrier()
    pltpu.sync_copy(acc_sh.at[pl.ds(z0, ZROWS)],
                    out_hbm.at[c, pl.ds(z0, ZROWS)])

  return k


def _sc_scatter_edges(*args):
  return _make_sc_scatter(E_CHUNKS)(*args)


def _sc_scatter_tokens(*args):
  return _make_sc_scatter(TOK_CHUNKS)(*args)


_BR = NODE_P // 16
_NRB = NODE_P // _BR


def _mm_body(s_ref, w_ref, b_ref, o_ref):
  t = pl.program_id(1)
  ch = pl.program_id(2)
  x = jnp.concatenate([s_ref[0], s_ref[1]], axis=1)
  w = w_ref[ch, t]
  o_ref[0, 0] = jnp.dot(x, w, preferred_element_type=jnp.float32) + b_ref[ch, t]


def _msg_transform(states_h, w3, b3):
  return pl.pallas_call(
      _mm_body,
      grid=(_NRB, N_TYPES, NC),
      in_specs=[
          pl.BlockSpec((NC, _BR, HALF), lambda rb, t, ch: (0, rb, 0)),
          pl.BlockSpec((NC, N_TYPES, HIDDEN, HALF), lambda rb, t, ch: (0, 0, 0, 0)),
          pl.BlockSpec((NC, N_TYPES, HALF), lambda rb, t, ch: (0, 0, 0)),
      ],
      out_specs=pl.BlockSpec((1, 1, _BR, HALF), lambda rb, t, ch: (ch, t, rb, 0)),
      out_shape=jax.ShapeDtypeStruct((NC, N_TYPES, NODE_P, HALF), jnp.float32),
  )(states_h, w3, b3)


def _gru_body(a_ref, s_ref, gk_ref, gb_ref, ck_ref, cb_ref, o_ref):
  a = jnp.concatenate([a_ref[0], a_ref[1]], axis=1)
  st = jnp.concatenate([s_ref[0], s_ref[1]], axis=1)
  gi = jnp.concatenate([a, st], axis=1)
  gates = jax.nn.sigmoid(
      jnp.dot(gi, gk_ref[...], preferred_element_type=jnp.float32) + gb_ref[0])
  r = gates[:, :HIDDEN]
  u = gates[:, HIDDEN:]
  ci = jnp.concatenate([a, r * st], axis=1)
  cand = jnp.tanh(
      jnp.dot(ci, ck_ref[...], preferred_element_type=jnp.float32) + cb_ref[0])
  new = u * st + (1.0 - u) * cand
  o_ref[0] = new[:, :HALF]
  o_ref[1] = new[:, HALF:]


def _gru(agg_h, states_h, gk, gb, ck, cb):
  blk = pl.BlockSpec((NC, _BR, HALF), lambda rb: (0, rb, 0))
  return pl.pallas_call(
      _gru_body,
      grid=(_NRB,),
      in_specs=[
          blk, blk,
          pl.BlockSpec((2 * HIDDEN, 2 * HIDDEN), lambda rb: (0, 0)),
          pl.BlockSpec((1, 2 * HIDDEN), lambda rb: (0, 0)),
          pl.BlockSpec((2 * HIDDEN, HIDDEN), lambda rb: (0, 0)),
          pl.BlockSpec((1, HIDDEN), lambda rb: (0, 0)),
      ],
      out_specs=blk,
      out_shape=jax.ShapeDtypeStruct((NC, NODE_P, HALF), jnp.float32),
  )(agg_h, states_h, gk, gb, ck, cb)


def kernel(node_indices, node_segment_ids, edge_sources, edge_targets,
           embedding, type_weights, type_biases,
           gru_gate_kernel, gru_gate_bias, gru_cand_kernel, gru_cand_bias):
  i32 = jnp.int32
  emb_flat = jnp.stack([embedding[:, :HALF], embedding[:, HALF:]]).reshape(2 * VOCAB, HALF)

  src_tok = jnp.concatenate(
      [node_indices.astype(i32), jnp.zeros((TOK_PAD - N_TOKENS,), i32)])
  src2_tok = jnp.concatenate([src_tok, src_tok + VOCAB])
  tgt_tok = jnp.concatenate(
      [node_segment_ids.astype(i32),
       jnp.full((TOK_PAD - N_TOKENS,), _DUMMY_TGT, i32)])

  src_e = (edge_sources.astype(i32)
           + (jnp.arange(N_TYPES, dtype=i32) * NODE_P)[:, None]).reshape(-1)
  src_e = jnp.concatenate([src_e, jnp.zeros((E_PAD - N_TYPES * EDGES_PER_TYPE,), i32)])
  src2_e = jnp.concatenate([src_e, src_e + N_TYPES * NODE_P])
  tgt_e = jnp.concatenate(
      [edge_targets.astype(i32).reshape(-1),
       jnp.full((E_PAD - N_TYPES * EDGES_PER_TYPE,), _DUMMY_TGT, i32)])

  zero_sp = jnp.zeros((NODE_P, HALF), jnp.float32)

  states_h = _sc_scatter_tokens(emb_flat, src2_tok, tgt_tok, zero_sp)

  for layer, steps in enumerate(TIME_STEPS):
    w3 = type_weights[layer].reshape(N_TYPES, HIDDEN, NC, HALF).transpose(2, 0, 1, 3)
    b3 = type_biases[layer].reshape(N_TYPES, NC, HALF).transpose(1, 0, 2)
    gk = gru_gate_kernel[layer]
    gb = gru_gate_bias[layer].reshape(1, 2 * HIDDEN)
    ck = gru_cand_kernel[layer]
    cb = gru_cand_bias[layer].reshape(1, HIDDEN)
    for _ in range(steps):
      t_tab = _msg_transform(states_h, w3, b3).reshape(2 * N_TYPES * NODE_P, HALF)
      agg_h = _sc_scatter_edges(t_tab, src2_e, tgt_e, zero_sp)
      states_h = _gru(agg_h, states_h, gk, gb, ck, cb)

  return jnp.concatenate([states_h[0], states_h[1]], axis=1)[:N_NODES]


# speedup vs baseline: 1.0185x; 1.0185x over previous
import functools

import jax
import jax.numpy as jnp
from jax import lax
from jax.experimental import pallas as pl
from jax.experimental.pallas import tpu as pltpu
from jax.experimental.pallas import tpu_sc as plsc

N_NODES = 10000
HIDDEN = 256
HALF = 128
VOCAB = 5000
N_TYPES = 4
EDGES_PER_TYPE = 40000
N_TOKENS = 20000
TIME_STEPS = [3, 3]

NC = 2
NS = 16
CHUNK = 128

NODE_P = 10112
ZROWS = NODE_P // NS

E_CHUNKS = 80
TOK_CHUNKS = 16
E_PAD = NS * E_CHUNKS * CHUNK
TOK_PAD = NS * TOK_CHUNKS * CHUNK
NBUF = 2
NIDX = 4

_DUMMY_TGT = N_NODES


@functools.lru_cache(maxsize=None)
def _make_sc_scatter(n_chunks):
  mesh = plsc.VectorSubcoreMesh(core_axis_name="c", subcore_axis_name="s",
                                num_cores=NC, num_subcores=NS)
  per_tile = n_chunks * CHUNK

  @functools.partial(
      pl.kernel,
      out_type=jax.ShapeDtypeStruct((NC, NODE_P, HALF), jnp.float32),
      mesh=mesh,
      scratch_types=[
          pltpu.VMEM((NIDX, CHUNK), jnp.int32),
          pltpu.VMEM((NIDX, CHUNK), jnp.int32),
          pltpu.VMEM((NBUF, CHUNK, HALF), jnp.float32),
          pltpu.VMEM_SHARED((NODE_P, HALF), jnp.float32),
          pltpu.SemaphoreType.DMA((NBUF,)),
          pltpu.SemaphoreType.DMA((NBUF,)),
          pltpu.SemaphoreType.DMA((NIDX,)),
      ],
  )
  def k(table_hbm, src_hbm, tgt_hbm, zero_hbm, out_hbm,
        src_v, tgt_v, rows_v, acc_sh, sem_g, sem_s, sem_i):
    c = lax.axis_index("c")
    s = lax.axis_index("s")
    sbase = (c * NS + s) * per_tile
    tbase = s * per_tile

    def load_idx(j, bi):
      pltpu.async_copy(src_hbm.at[pl.ds(sbase + j * CHUNK, CHUNK)],
                       src_v.at[bi], sem_i.at[bi])
      pltpu.async_copy(tgt_hbm.at[pl.ds(tbase + j * CHUNK, CHUNK)],
                       tgt_v.at[bi], sem_i.at[bi])

    def wait_idx(bi):
      pltpu.make_async_copy(src_hbm.at[pl.ds(sbase, CHUNK)], src_v.at[bi],
                            sem_i.at[bi]).wait()
      pltpu.make_async_copy(tgt_hbm.at[pl.ds(tbase, CHUNK)], tgt_v.at[bi],
                            sem_i.at[bi]).wait()

    def gather(j, b, bi):
      pltpu.async_copy(table_hbm.at[src_v.at[bi]], rows_v.at[b], sem_g.at[b])

    def wait_gather(b):
      pltpu.make_async_copy(table_hbm.at[src_v.at[0]], rows_v.at[b],
                            sem_g.at[b]).wait()

    def scatter(b, bi):
      pltpu.async_copy(rows_v.at[b], acc_sh.at[tgt_v.at[bi]], sem_s.at[b],
                       add=True)

    def wait_scatter(b):
      pltpu.make_async_copy(rows_v.at[b], acc_sh.at[tgt_v.at[0]],
                            sem_s.at[b]).wait()

    for j in range(NIDX):
      load_idx(j, j)
    z0 = s * ZROWS
    pltpu.sync_copy(zero_hbm.at[pl.ds(z0, ZROWS)], acc_sh.at[pl.ds(z0, ZROWS)])
    plsc.subcore_barrier()

    for j in range(NBUF):
      wait_idx(j)
      gather(j, j, j)

    def quad_body(it, carry):
      kk = it * NIDX
      for i in range(NIDX):
        kc = kk + i
        b = i % NBUF
        bi = i
        wait_gather(b)
        scatter(b, bi)
        j = kc + 2
        bj = (i + 2) % NBUF
        bij = (i + 2) % NIDX

        @pl.when(j < n_chunks)
        def _():
          wait_scatter(bj)
          @pl.when(j + 2 < n_chunks)
          def _():
            load_idx(j + 2, bi)
          wait_idx(bij)
          gather(j, bj, bij)
      return carry

    lax.fori_loop(0, n_chunks // NIDX, quad_body, 0)
    for b in range(NBUF):
      wait_scatter(b)
    plsc.subcore_barrier()
    pltpu.sync_copy(acc_sh.at[pl.ds(z0, ZROWS)],
                    out_hbm.at[c, pl.ds(z0, ZROWS)])

  return k


def _sc_scatter_edges(*args):
  return _make_sc_scatter(E_CHUNKS)(*args)


def _sc_scatter_tokens(*args):
  return _make_sc_scatter(TOK_CHUNKS)(*args)


_BR = NODE_P // 16
_NRB = NODE_P // _BR


def _mm_body(s_ref, w_ref, b_ref, o_ref):
  t = pl.program_id(1)
  ch = pl.program_id(2)
  x = jnp.concatenate([s_ref[0], s_ref[1]], axis=1)
  w = w_ref[ch, t]
  o_ref[0, 0] = jnp.dot(x, w, preferred_element_type=jnp.float32) + b_ref[ch, t]


def _msg_transform(states_h, w3, b3):
  return pl.pallas_call(
      _mm_body,
      grid=(_NRB, N_TYPES, NC),
      in_specs=[
          pl.BlockSpec((NC, _BR, HALF), lambda rb, t, ch: (0, rb, 0)),
          pl.BlockSpec((NC, N_TYPES, HIDDEN, HALF), lambda rb, t, ch: (0, 0, 0, 0)),
          pl.BlockSpec((NC, N_TYPES, HALF), lambda rb, t, ch: (0, 0, 0)),
      ],
      out_specs=pl.BlockSpec((1, 1, _BR, HALF), lambda rb, t, ch: (ch, t, rb, 0)),
      out_shape=jax.ShapeDtypeStruct((NC, N_TYPES, NODE_P, HALF), jnp.float32),
  )(states_h, w3, b3)


def _gru_body(a_ref, s_ref, gk_ref, gb_ref, ck_ref, cb_ref, o_ref):
  a = jnp.concatenate([a_ref[0], a_ref[1]], axis=1)
  st = jnp.concatenate([s_ref[0], s_ref[1]], axis=1)
  gi = jnp.concatenate([a, st], axis=1)
  gates = jax.nn.sigmoid(
      jnp.dot(gi, gk_ref[...], preferred_element_type=jnp.float32) + gb_ref[0])
  r = gates[:, :HIDDEN]
  u = gates[:, HIDDEN:]
  ci = jnp.concatenate([a, r * st], axis=1)
  cand = jnp.tanh(
      jnp.dot(ci, ck_ref[...], preferred_element_type=jnp.float32) + cb_ref[0])
  new = u * st + (1.0 - u) * cand
  o_ref[0] = new[:, :HALF]
  o_ref[1] = new[:, HALF:]


def _gru(agg_h, states_h, gk, gb, ck, cb):
  blk = pl.BlockSpec((NC, _BR, HALF), lambda rb: (0, rb, 0))
  return pl.pallas_call(
      _gru_body,
      grid=(_NRB,),
      in_specs=[
          blk, blk,
          pl.BlockSpec((2 * HIDDEN, 2 * HIDDEN), lambda rb: (0, 0)),
          pl.BlockSpec((1, 2 * HIDDEN), lambda rb: (0, 0)),
          pl.BlockSpec((2 * HIDDEN, HIDDEN), lambda rb: (0, 0)),
          pl.BlockSpec((1, HIDDEN), lambda rb: (0, 0)),
      ],
      out_specs=blk,
      out_shape=jax.ShapeDtypeStruct((NC, NODE_P, HALF), jnp.float32),
  )(agg_h, states_h, gk, gb, ck, cb)


def kernel(node_indices, node_segment_ids, edge_sources, edge_targets,
           embedding, type_weights, type_biases,
           gru_gate_kernel, gru_gate_bias, gru_cand_kernel, gru_cand_bias):
  i32 = jnp.int32
  emb_flat = jnp.stack([embedding[:, :HALF], embedding[:, HALF:]]).reshape(2 * VOCAB, HALF)

  src_tok = jnp.concatenate(
      [node_indices.astype(i32), jnp.zeros((TOK_PAD - N_TOKENS,), i32)])
  src2_tok = jnp.concatenate([src_tok, src_tok + VOCAB])
  tgt_tok = jnp.concatenate(
      [node_segment_ids.astype(i32),
       jnp.full((TOK_PAD - N_TOKENS,), _DUMMY_TGT, i32)])

  src_e = (edge_sources.astype(i32)
           + (jnp.arange(N_TYPES, dtype=i32) * NODE_P)[:, None]).reshape(-1)
  src_e = jnp.concatenate([src_e, jnp.zeros((E_PAD - N_TYPES * EDGES_PER_TYPE,), i32)])
  src2_e = jnp.concatenate([src_e, src_e + N_TYPES * NODE_P])
  tgt_e = jnp.concatenate(
      [edge_targets.astype(i32).reshape(-1),
       jnp.full((E_PAD - N_TYPES * EDGES_PER_TYPE,), _DUMMY_TGT, i32)])

  zero_sp = jnp.zeros((NODE_P, HALF), jnp.float32)

  states_h = _sc_scatter_tokens(emb_flat, src2_tok, tgt_tok, zero_sp)

  for layer, steps in enumerate(TIME_STEPS):
    w3 = type_weights[layer].reshape(N_TYPES, HIDDEN, NC, HALF).transpose(2, 0, 1, 3)
    b3 = type_biases[layer].reshape(N_TYPES, NC, HALF).transpose(1, 0, 2)
    gk = gru_gate_kernel[layer]
    gb = gru_gate_bias[layer].reshape(1, 2 * HIDDEN)
    ck = gru_cand_kernel[layer]
    cb = gru_cand_bias[layer].reshape(1, HIDDEN)
    for _ in range(steps):
      t_tab = _msg_transform(states_h, w3, b3).reshape(2 * N_TYPES * NODE_P, HALF)
      agg_h = _sc_scatter_edges(t_tab, src2_e, tgt_e, zero_sp)
      states_h = _gru(agg_h, states_h, gk, gb, ck, cb)

  return jnp.concatenate([states_h[0], states_h[1]], axis=1)[:N_NODES]


# speedup vs baseline: 1.4952x; 1.4681x over previous
import functools

import jax
import jax.numpy as jnp
from jax import lax
from jax.experimental import pallas as pl
from jax.experimental.pallas import tpu as pltpu
from jax.experimental.pallas import tpu_sc as plsc

N_NODES = 10000
HIDDEN = 256
VOCAB = 5000
N_TYPES = 4
EDGES_PER_TYPE = 40000
N_EDGES = N_TYPES * EDGES_PER_TYPE
N_TOKENS = 20000
TIME_STEPS = [3, 3]

NC = 2
NS = 16
CHUNK = 64
NBUF = 2
NIDX = 4

NODE_P = 10112
NHALF = NODE_P // 2
ACC_R = 5120
ZROWS = ACC_R // NS

E_CHUNKS = 112
E_CAP = NS * E_CHUNKS * CHUNK
E_OVL = N_EDGES - E_CAP
TOK_PAD = 20480
TOK_CHUNKS = 16
TOK_CAP = NS * TOK_CHUNKS * CHUNK
TOK_OVL = TOK_PAD - TOK_CAP

_DUMMY = NHALF


@functools.lru_cache(maxsize=None)
def _make_sc_scatter(n_chunks):
  mesh = plsc.VectorSubcoreMesh(core_axis_name="c", subcore_axis_name="s",
                                num_cores=NC, num_subcores=NS)
  per_tile = n_chunks * CHUNK
  e_len = NS * per_tile

  @functools.partial(
      pl.kernel,
      out_type=jax.ShapeDtypeStruct((NODE_P, 2, 128), jnp.float32),
      mesh=mesh,
      scratch_types=[
          pltpu.VMEM((NIDX, CHUNK), jnp.int32),
          pltpu.VMEM((NIDX, CHUNK), jnp.int32),
          pltpu.VMEM((NBUF, CHUNK, 2, 128), jnp.float32),
          pltpu.VMEM_SHARED((ACC_R, 2, 128), jnp.float32),
          pltpu.SemaphoreType.DMA((NBUF,)),
          pltpu.SemaphoreType.DMA((NBUF,)),
          pltpu.SemaphoreType.DMA((NIDX,)),
      ],
  )
  def k(table_hbm, src_hbm, tgt_hbm, zero_hbm, out_hbm,
        src_v, tgt_v, rows_v, acc_sh, sem_g, sem_s, sem_i):
    c = lax.axis_index("c")
    s = lax.axis_index("s")
    base = c * e_len + s * per_tile

    def load_idx(j, bi):
      pltpu.async_copy(src_hbm.at[pl.ds(base + j * CHUNK, CHUNK)],
                       src_v.at[bi], sem_i.at[bi])
      pltpu.async_copy(tgt_hbm.at[pl.ds(base + j * CHUNK, CHUNK)],
                       tgt_v.at[bi], sem_i.at[bi])

    def wait_idx(bi):
      pltpu.make_async_copy(src_hbm.at[pl.ds(base, CHUNK)], src_v.at[bi],
                            sem_i.at[bi]).wait()
      pltpu.make_async_copy(tgt_hbm.at[pl.ds(base, CHUNK)], tgt_v.at[bi],
                            sem_i.at[bi]).wait()

    def gather(b, bi):
      pltpu.async_copy(table_hbm.at[src_v.at[bi]], rows_v.at[b], sem_g.at[b])

    def wait_gather(b):
      pltpu.make_async_copy(table_hbm.at[src_v.at[0]], rows_v.at[b],
                            sem_g.at[b]).wait()

    def scatter(b, bi):
      pltpu.async_copy(rows_v.at[b], acc_sh.at[tgt_v.at[bi]], sem_s.at[b],
                       add=True)

    def wait_scatter(b):
      pltpu.make_async_copy(rows_v.at[b], acc_sh.at[tgt_v.at[0]],
                            sem_s.at[b]).wait()

    for j in range(NIDX):
      load_idx(j, j)
    z0 = s * ZROWS
    pltpu.sync_copy(zero_hbm.at[pl.ds(z0, ZROWS)], acc_sh.at[pl.ds(z0, ZROWS)])
    plsc.subcore_barrier()

    for j in range(NBUF):
      wait_idx(j)
      gather(j, j)

    def quad_body(it, carry):
      for i in range(NIDX):
        kc = it * NIDX + i
        b = i % NBUF
        wait_gather(b)
        scatter(b, i)
        j = kc + 2
        bj = (i + 2) % NBUF
        bij = (i + 2) % NIDX

        @pl.when(j < n_chunks)
        def _():
          wait_scatter(bj)
          @pl.when(j + 2 < n_chunks)
          def _():
            load_idx(j + 2, i)
          wait_idx(bij)
          gather(bj, bij)
      return carry

    lax.fori_loop(0, n_chunks // NIDX, quad_body, 0)
    for b in range(NBUF):
      wait_scatter(b)
    plsc.subcore_barrier()
    o0 = c * NHALF + z0

    @pl.when(s < NS - 1)
    def _():
      pltpu.sync_copy(acc_sh.at[pl.ds(z0, ZROWS)], out_hbm.at[pl.ds(o0, ZROWS)])

    @pl.when(s == NS - 1)
    def _():
      pltpu.sync_copy(acc_sh.at[pl.ds(z0, NHALF - (NS - 1) * ZROWS)],
                      out_hbm.at[pl.ds(o0, NHALF - (NS - 1) * ZROWS)])

  return k


def _sc_scatter_edges(*args):
  return _make_sc_scatter(E_CHUNKS)(*args)


def _sc_scatter_tokens(*args):
  return _make_sc_scatter(TOK_CHUNKS)(*args)


_BR = NODE_P // 16
_NRB = NODE_P // _BR


def _mm_body(s_ref, w_ref, b_ref, o_ref):
  t = pl.program_id(1)
  o_ref[0] = (jnp.dot(s_ref[...], w_ref[t], preferred_element_type=jnp.float32)
              + b_ref[t])


def _msg_transform(states, w, b):
  return pl.pallas_call(
      _mm_body,
      grid=(_NRB, N_TYPES),
      in_specs=[
          pl.BlockSpec((_BR, HIDDEN), lambda rb, t: (rb, 0)),
          pl.BlockSpec((N_TYPES, HIDDEN, HIDDEN), lambda rb, t: (0, 0, 0)),
          pl.BlockSpec((N_TYPES, HIDDEN), lambda rb, t: (0, 0)),
      ],
      out_specs=pl.BlockSpec((1, _BR, HIDDEN), lambda rb, t: (t, rb, 0)),
      out_shape=jax.ShapeDtypeStruct((N_TYPES, NODE_P, HIDDEN), jnp.float32),
  )(states, w, b)


def _gru_body(a_ref, s_ref, gk_ref, gb_ref, ck_ref, cb_ref, o_ref):
  a = a_ref[...]
  st = s_ref[...]
  gi = jnp.concatenate([a, st], axis=1)
  gates = jax.nn.sigmoid(
      jnp.dot(gi, gk_ref[...], preferred_element_type=jnp.float32) + gb_ref[0])
  r = gates[:, :HIDDEN]
  u = gates[:, HIDDEN:]
  ci = jnp.concatenate([a, r * st], axis=1)
  cand = jnp.tanh(
      jnp.dot(ci, ck_ref[...], preferred_element_type=jnp.float32) + cb_ref[0])
  o_ref[...] = u * st + (1.0 - u) * cand


def _gru(agg, states, gk, gb, ck, cb):
  blk = pl.BlockSpec((_BR, HIDDEN), lambda rb: (rb, 0))
  return pl.pallas_call(
      _gru_body,
      grid=(_NRB,),
      in_specs=[
          blk, blk,
          pl.BlockSpec((2 * HIDDEN, 2 * HIDDEN), lambda rb: (0, 0)),
          pl.BlockSpec((1, 2 * HIDDEN), lambda rb: (0, 0)),
          pl.BlockSpec((2 * HIDDEN, HIDDEN), lambda rb: (0, 0)),
          pl.BlockSpec((1, HIDDEN), lambda rb: (0, 0)),
      ],
      out_specs=blk,
      out_shape=jax.ShapeDtypeStruct((NODE_P, HIDDEN), jnp.float32),
  )(agg, states, gk, gb, ck, cb)


def _split_lists(src, tgt, cap, ovl):
  src0, tgt0 = src[:cap], tgt[:cap]
  src1, tgt1 = src[ovl:ovl + cap], tgt[ovl:ovl + cap]
  loc0 = jnp.where(tgt0 < NHALF, tgt0, _DUMMY)
  loc1 = jnp.where(tgt1 >= NHALF, tgt1 - NHALF, _DUMMY)
  return jnp.concatenate([src0, src1]), jnp.concatenate([loc0, loc1])


def kernel(node_indices, node_segment_ids, edge_sources, edge_targets,
           embedding, type_weights, type_biases,
           gru_gate_kernel, gru_gate_bias, gru_cand_kernel, gru_cand_bias):
  i32 = jnp.int32
  emb3 = embedding.reshape(VOCAB, 2, 128)

  src_tok = jnp.concatenate(
      [node_indices.astype(i32), jnp.zeros((TOK_PAD - N_TOKENS,), i32)])
  tgt_tok = jnp.concatenate(
      [node_segment_ids.astype(i32),
       jnp.full((TOK_PAD - N_TOKENS,), N_NODES, i32)])
  src2_tok, tgt2_tok = _split_lists(src_tok, tgt_tok, TOK_CAP, TOK_OVL)

  src_e = (edge_sources.astype(i32)
           + (jnp.arange(N_TYPES, dtype=i32) * NODE_P)[:, None]).reshape(-1)
  tgt_e = edge_targets.astype(i32).reshape(-1)
  _, src_s, tgt_s = lax.sort(
      [(tgt_e >= NHALF).astype(i32), src_e, tgt_e], num_keys=1)
  src2_e, tgt2_e = _split_lists(src_s, tgt_s, E_CAP, E_OVL)

  zero_sp = jnp.zeros((ACC_R, 2, 128), jnp.float32)

  states = _sc_scatter_tokens(emb3, src2_tok, tgt2_tok, zero_sp)
  states = states.reshape(NODE_P, HIDDEN)

  for layer, steps in enumerate(TIME_STEPS):
    w = type_weights[layer]
    b = type_biases[layer]
    gk = gru_gate_kernel[layer]
    gb = gru_gate_bias[layer].reshape(1, 2 * HIDDEN)
    ck = gru_cand_kernel[layer]
    cb = gru_cand_bias[layer].reshape(1, HIDDEN)
    for _ in range(steps):
      t_tab = _msg_transform(states, w, b).reshape(N_TYPES * NODE_P, 2, 128)
      agg = _sc_scatter_edges(t_tab, src2_e, tgt2_e, zero_sp)
      states = _gru(agg.reshape(NODE_P, HIDDEN), states, gk, gb, ck, cb)

  return states[:N_NODES]


# speedup vs baseline: 1.4975x; 1.0015x over previous
import functools

import jax
import jax.numpy as jnp
from jax import lax
from jax.experimental import pallas as pl
from jax.experimental.pallas import tpu as pltpu
from jax.experimental.pallas import tpu_sc as plsc

N_NODES = 10000
HIDDEN = 256
VOCAB = 5000
N_TYPES = 4
EDGES_PER_TYPE = 40000
N_EDGES = N_TYPES * EDGES_PER_TYPE
N_TOKENS = 20000
TIME_STEPS = [3, 3]

NC = 2
NS = 16
CHUNK = 64
NBUF = 2
NIDX = 4

NODE_P = 10112
NHALF = NODE_P // 2
ACC_R = 5120
ZROWS = ACC_R // NS

E_CHUNKS = 112
E_CAP = NS * E_CHUNKS * CHUNK
E_OVL = N_EDGES - E_CAP
TOK_PAD = 20480
TOK_CHUNKS = 16
TOK_CAP = NS * TOK_CHUNKS * CHUNK
TOK_OVL = TOK_PAD - TOK_CAP

_DUMMY = NHALF


@functools.lru_cache(maxsize=None)
def _make_sc_scatter(n_chunks):
  mesh = plsc.VectorSubcoreMesh(core_axis_name="c", subcore_axis_name="s",
                                num_cores=NC, num_subcores=NS)
  per_tile = n_chunks * CHUNK
  e_len = NS * per_tile

  @functools.partial(
      pl.kernel,
      out_type=jax.ShapeDtypeStruct((NC, NHALF, 2, 128), jnp.float32),
      mesh=mesh,
      scratch_types=[
          pltpu.VMEM((NIDX, CHUNK), jnp.int32),
          pltpu.VMEM((NIDX, CHUNK), jnp.int32),
          pltpu.VMEM((NBUF, CHUNK, 2, 128), jnp.float32),
          pltpu.VMEM_SHARED((ACC_R, 2, 128), jnp.float32),
          pltpu.SemaphoreType.DMA((NBUF,)),
          pltpu.SemaphoreType.DMA((NBUF,)),
          pltpu.SemaphoreType.DMA((NIDX,)),
      ],
  )
  def k(table_hbm, src_hbm, tgt_hbm, zero_hbm, out_hbm,
        src_v, tgt_v, rows_v, acc_sh, sem_g, sem_s, sem_i):
    c = lax.axis_index("c")
    s = lax.axis_index("s")
    base = c * e_len + s * per_tile

    def load_idx(j, bi):
      pltpu.async_copy(src_hbm.at[pl.ds(base + j * CHUNK, CHUNK)],
                       src_v.at[bi], sem_i.at[bi])
      pltpu.async_copy(tgt_hbm.at[pl.ds(base + j * CHUNK, CHUNK)],
                       tgt_v.at[bi], sem_i.at[bi])

    def wait_idx(bi):
      pltpu.make_async_copy(src_hbm.at[pl.ds(base, CHUNK)], src_v.at[bi],
                            sem_i.at[bi]).wait()
      pltpu.make_async_copy(tgt_hbm.at[pl.ds(base, CHUNK)], tgt_v.at[bi],
                            sem_i.at[bi]).wait()

    def gather(b, bi):
      pltpu.async_copy(table_hbm.at[src_v.at[bi]], rows_v.at[b], sem_g.at[b])

    def wait_gather(b):
      pltpu.make_async_copy(table_hbm.at[src_v.at[0]], rows_v.at[b],
                            sem_g.at[b]).wait()

    def scatter(b, bi):
      pltpu.async_copy(rows_v.at[b], acc_sh.at[tgt_v.at[bi]], sem_s.at[b],
                       add=True)

    def wait_scatter(b):
      pltpu.make_async_copy(rows_v.at[b], acc_sh.at[tgt_v.at[0]],
                            sem_s.at[b]).wait()

    for j in range(NIDX):
      load_idx(j, j)
    z0 = s * ZROWS
    pltpu.sync_copy(zero_hbm.at[pl.ds(z0, ZROWS)], acc_sh.at[pl.ds(z0, ZROWS)])
    plsc.subcore_barrier()

    for j in range(NBUF):
      wait_idx(j)
      gather(j, j)

    def quad_body(it, carry):
      for i in range(NIDX):
        kc = it * NIDX + i
        b = i % NBUF
        wait_gather(b)
        scatter(b, i)
        j = kc + 2
        bj = (i + 2) % NBUF
        bij = (i + 2) % NIDX

        @pl.when(j < n_chunks)
        def _():
          wait_scatter(bj)
          @pl.when(j + 2 < n_chunks)
          def _():
            load_idx(j + 2, i)
          wait_idx(bij)
          gather(bj, bij)
      return carry

    lax.fori_loop(0, n_chunks // NIDX, quad_body, 0)
    for b in range(NBUF):
      wait_scatter(b)
    plsc.subcore_barrier()
    @pl.when(s < NS - 1)
    def _():
      pltpu.sync_copy(acc_sh.at[pl.ds(z0, ZROWS)],
                      out_hbm.at[c, pl.ds(z0, ZROWS)])

    @pl.when(s == NS - 1)
    def _():
      pltpu.sync_copy(acc_sh.at[pl.ds(z0, NHALF - (NS - 1) * ZROWS)],
                      out_hbm.at[c, pl.ds(z0, NHALF - (NS - 1) * ZROWS)])

  return k


def _sc_scatter_edges(*args):
  return _make_sc_scatter(E_CHUNKS)(*args)


def _sc_scatter_tokens(*args):
  return _make_sc_scatter(TOK_CHUNKS)(*args)


_BR = NODE_P // 16
_NRB = NODE_P // _BR


def _mm_body(s_ref, w_ref, b_ref, o_ref):
  t = pl.program_id(1)
  o_ref[0] = (jnp.dot(s_ref[...], w_ref[t], preferred_element_type=jnp.float32)
              + b_ref[t])


def _msg_transform(states, w, b):
  return pl.pallas_call(
      _mm_body,
      grid=(_NRB, N_TYPES),
      in_specs=[
          pl.BlockSpec((_BR, HIDDEN), lambda rb, t: (rb, 0)),
          pl.BlockSpec((N_TYPES, HIDDEN, HIDDEN), lambda rb, t: (0, 0, 0)),
          pl.BlockSpec((N_TYPES, HIDDEN), lambda rb, t: (0, 0)),
      ],
      out_specs=pl.BlockSpec((1, _BR, HIDDEN), lambda rb, t: (t, rb, 0)),
      out_shape=jax.ShapeDtypeStruct((N_TYPES, NODE_P, HIDDEN), jnp.float32),
  )(states, w, b)


def _gru_body(a_ref, s_ref, gk_ref, gb_ref, ck_ref, cb_ref, o_ref):
  a = a_ref[...]
  st = s_ref[...]
  gi = jnp.concatenate([a, st], axis=1)
  gates = jax.nn.sigmoid(
      jnp.dot(gi, gk_ref[...], preferred_element_type=jnp.float32) + gb_ref[0])
  r = gates[:, :HIDDEN]
  u = gates[:, HIDDEN:]
  ci = jnp.concatenate([a, r * st], axis=1)
  cand = jnp.tanh(
      jnp.dot(ci, ck_ref[...], preferred_element_type=jnp.float32) + cb_ref[0])
  o_ref[...] = u * st + (1.0 - u) * cand


def _gru(agg, states, gk, gb, ck, cb):
  blk = pl.BlockSpec((_BR, HIDDEN), lambda rb: (rb, 0))
  return pl.pallas_call(
      _gru_body,
      grid=(_NRB,),
      in_specs=[
          blk, blk,
          pl.BlockSpec((2 * HIDDEN, 2 * HIDDEN), lambda rb: (0, 0)),
          pl.BlockSpec((1, 2 * HIDDEN), lambda rb: (0, 0)),
          pl.BlockSpec((2 * HIDDEN, HIDDEN), lambda rb: (0, 0)),
          pl.BlockSpec((1, HIDDEN), lambda rb: (0, 0)),
      ],
      out_specs=blk,
      out_shape=jax.ShapeDtypeStruct((NODE_P, HIDDEN), jnp.float32),
  )(agg, states, gk, gb, ck, cb)


def _split_lists(src, tgt, cap, ovl):
  src0, tgt0 = src[:cap], tgt[:cap]
  src1, tgt1 = src[ovl:ovl + cap], tgt[ovl:ovl + cap]
  loc0 = jnp.where(tgt0 < NHALF, tgt0, _DUMMY)
  loc1 = jnp.where(tgt1 >= NHALF, tgt1 - NHALF, _DUMMY)
  return jnp.concatenate([src0, src1]), jnp.concatenate([loc0, loc1])


def kernel(node_indices, node_segment_ids, edge_sources, edge_targets,
           embedding, type_weights, type_biases,
           gru_gate_kernel, gru_gate_bias, gru_cand_kernel, gru_cand_bias):
  i32 = jnp.int32
  emb3 = embedding.reshape(VOCAB, 2, 128)

  src_tok = jnp.concatenate(
      [node_indices.astype(i32), jnp.zeros((TOK_PAD - N_TOKENS,), i32)])
  tgt_tok = jnp.concatenate(
      [node_segment_ids.astype(i32),
       jnp.full((TOK_PAD - N_TOKENS,), N_NODES, i32)])
  src2_tok, tgt2_tok = _split_lists(src_tok, tgt_tok, TOK_CAP, TOK_OVL)

  src_e = (edge_sources.astype(i32)
           + (jnp.arange(N_TYPES, dtype=i32) * NODE_P)[:, None]).reshape(-1)
  tgt_e = edge_targets.astype(i32).reshape(-1)
  _, src_s, tgt_s = lax.sort(
      [(tgt_e >= NHALF).astype(i32), src_e, tgt_e], num_keys=1)
  src2_e, tgt2_e = _split_lists(src_s, tgt_s, E_CAP, E_OVL)

  zero_sp = jnp.zeros((ACC_R, 2, 128), jnp.float32)

  states = _sc_scatter_tokens(emb3, src2_tok, tgt2_tok, zero_sp)
  states = states.reshape(NODE_P, HIDDEN)

  for layer, steps in enumerate(TIME_STEPS):
    w = type_weights[layer]
    b = type_biases[layer]
    gk = gru_gate_kernel[layer]
    gb = gru_gate_bias[layer].reshape(1, 2 * HIDDEN)
    ck = gru_cand_kernel[layer]
    cb = gru_cand_bias[layer].reshape(1, HIDDEN)
    for _ in range(steps):
      t_tab = _msg_transform(states, w, b).reshape(N_TYPES * NODE_P, 2, 128)
      agg = _sc_scatter_edges(t_tab, src2_e, tgt2_e, zero_sp)
      states = _gru(agg.reshape(NODE_P, HIDDEN), states, gk, gb, ck, cb)


  return states[:N_NODES]
